# Initial kernel scaffold; baseline (speedup 1.0000x reference)
import jax
import jax.numpy as jnp
from jax.experimental import pallas as pl


def kernel(pred_contours, pred_offsets, gt_contours, gt_key_points, gt_key_points_mask):
    raise NotImplementedError("write your pallas kernel here")


# speedup vs baseline: 2.9428x; 2.9428x over previous
import jax
import jax.numpy as jnp
from jax import lax
from jax.experimental import pallas as pl
from jax.experimental.pallas import tpu as pltpu

_N = 128
_P = 128
_T = 10
_OFFSETS_STRIDE = 4.0
_KEY_ITEM_WEIGHT = 0.5
_IGNORE_BOUND = 1000.0
_BETA = 1.0 / _OFFSETS_STRIDE
_B = 8


def _smooth_l1(pred, target):
    diff = jnp.abs(pred - target)
    return jnp.where(diff < _BETA, 0.5 * diff * diff / _BETA, diff - 0.5 * _BETA)


def _dm_kernel(px, py, ox, oy, gx, gy, gxr, gyr, kx, ky, m, out_ref, acc):
    i = pl.program_id(0)

    @pl.when(i == 0)
    def _():
        acc[0] = 0.0
        acc[1] = 0.0
        acc[2] = 0.0
        acc[3] = 0.0

    pxv = px[...]
    pyv = py[...]
    oxv = ox[...]
    oyv = oy[...]
    gxv = gx[...]
    gyv = gy[...]
    gxrv = gxr[...]
    gyrv = gyr[...]
    kxv = kx[...]
    kyv = ky[...]
    mv = m[...]

    iota_j = lax.broadcasted_iota(jnp.int32, (_B, _P, _P), 1)
    pxb = pxv[:, None, :]
    pyb = pyv[:, None, :]

    runmin = jnp.full((_B, _P), jnp.inf, jnp.float32)
    seltx = jnp.zeros((_B, _P), jnp.float32)
    selty = jnp.zeros((_B, _P), jnp.float32)
    for s in range(_T):
        w = s / _T
        tsx = gxv * w + gxrv * (1.0 - w)
        tsy = gyv * w + gyrv * (1.0 - w)
        dx = tsx[:, :, None] - pxb
        dy = tsy[:, :, None] - pyb
        d = dx * dx + dy * dy
        mn = jnp.min(d, axis=1)
        first = jnp.min(jnp.where(d == mn[:, None, :], iota_j, _P), axis=1)
        onehot = iota_j == first[:, None, :]
        cx = jnp.sum(jnp.where(onehot, tsx[:, :, None], 0.0), axis=1)
        cy = jnp.sum(jnp.where(onehot, tsy[:, :, None], 0.0), axis=1)
        upd = mn < runmin
        runmin = jnp.where(upd, mn, runmin)
        seltx = jnp.where(upd, cx, seltx)
        selty = jnp.where(upd, cy, selty)

    valid1 = runmin <= _IGNORE_BOUND * _IGNORE_BOUND
    inv = 1.0 / _OFFSETS_STRIDE
    sl1 = _smooth_l1(oxv, (seltx - pxv) * inv) + _smooth_l1(oyv, (selty - pyv) * inv)
    acc[0] += jnp.sum(jnp.where(valid1, sl1, 0.0))
    acc[1] += jnp.sum(valid1.astype(jnp.float32))

    dx2 = pxv[:, :, None] - kxv[:, None, :]
    dy2 = pyv[:, :, None] - kyv[:, None, :]
    d2 = dx2 * dx2 + dy2 * dy2
    mn2 = jnp.min(d2, axis=1)
    firstp = jnp.min(jnp.where(d2 == mn2[:, None, :], iota_j, _P), axis=1)
    onehot2 = iota_j == firstp[:, None, :]
    pselx = jnp.sum(jnp.where(onehot2, pxv[:, :, None], 0.0), axis=1)
    psely = jnp.sum(jnp.where(onehot2, pyv[:, :, None], 0.0), axis=1)
    oselx = jnp.sum(jnp.where(onehot2, oxv[:, :, None], 0.0), axis=1)
    osely = jnp.sum(jnp.where(onehot2, oyv[:, :, None], 0.0), axis=1)

    valid2 = mn2 <= _IGNORE_BOUND * _IGNORE_BOUND
    m2 = jnp.logical_and(mv > 0.0, valid2)
    sl2 = _smooth_l1(oselx, (kxv - pselx) * inv) + _smooth_l1(osely, (kyv - psely) * inv)
    acc[2] += jnp.sum(jnp.where(m2, sl2, 0.0))
    acc[3] += jnp.sum(m2.astype(jnp.float32))

    @pl.when(i == pl.num_programs(0) - 1)
    def _():
        denom1 = jnp.maximum(acc[1] * 2.0, 1.0)
        denom2 = jnp.maximum(acc[3] * 2.0, 1.0)
        out_ref[0, 0] = (acc[0] / denom1) * (1.0 - _KEY_ITEM_WEIGHT) + (
            acc[2] / denom2
        ) * _KEY_ITEM_WEIGHT


def kernel(pred_contours, pred_offsets, gt_contours, gt_key_points, gt_key_points_mask):
    px = pred_contours[..., 0]
    py = pred_contours[..., 1]
    ox = pred_offsets[..., 0]
    oy = pred_offsets[..., 1]
    gx = gt_contours[..., 0]
    gy = gt_contours[..., 1]
    gxr = jnp.roll(gx, 1, axis=1)
    gyr = jnp.roll(gy, 1, axis=1)
    kx = gt_key_points[..., 0]
    ky = gt_key_points[..., 1]
    m = gt_key_points_mask.astype(jnp.float32)

    out = pl.pallas_call(
        _dm_kernel,
        grid=(_N // _B,),
        in_specs=[pl.BlockSpec((_B, _P), lambda i: (i, 0))] * 11,
        out_specs=pl.BlockSpec(memory_space=pltpu.SMEM),
        out_shape=jax.ShapeDtypeStruct((1, 1), jnp.float32),
        scratch_shapes=[pltpu.SMEM((4,), jnp.float32)],
    )(px, py, ox, oy, gx, gy, gxr, gyr, kx, ky, m)
    return out[0, 0]


# speedup vs baseline: 4.7787x; 1.6239x over previous
import jax
import jax.numpy as jnp
from jax import lax
from jax.experimental import pallas as pl
from jax.experimental.pallas import tpu as pltpu

_N = 128
_P = 128
_T = 10
_OFFSETS_STRIDE = 4.0
_KEY_ITEM_WEIGHT = 0.5
_IGNORE_BOUND = 1000.0
_BETA = 1.0 / _OFFSETS_STRIDE
_B = 8


def _smooth_l1(pred, target):
    diff = jnp.abs(pred - target)
    return jnp.where(diff < _BETA, 0.5 * diff * diff / _BETA, diff - 0.5 * _BETA)


def _dm_kernel(px, py, ox, oy, gx, gy, gxr, gyr, kx, ky, m, out_ref, acc):
    i = pl.program_id(0)

    @pl.when(i == 0)
    def _():
        acc[0] = 0.0
        acc[1] = 0.0
        acc[2] = 0.0
        acc[3] = 0.0

    pxv = px[...]
    pyv = py[...]
    oxv = ox[...]
    oyv = oy[...]
    gxv = gx[...]
    gyv = gy[...]
    gxrv = gxr[...]
    gyrv = gyr[...]
    kxv = kx[...]
    kyv = ky[...]
    mv = m[...]

    iota_j = lax.broadcasted_iota(jnp.int32, (_B, _P, _P), 1)

    bxj = gxv - gxrv
    byj = gyv - gyrv
    cj = bxj * bxj + byj * byj
    crj = jnp.where(cj > 1e-30, 1.0 / cj, 0.0)
    gbj = gxrv * bxj + gyrv * byj
    n2j = gxrv * gxrv + gyrv * gyrv
    p2p = pxv * pxv + pyv * pyv

    gp = gxrv[:, :, None] * pxv[:, None, :] + gyrv[:, :, None] * pyv[:, None, :]
    bp = bxj[:, :, None] * pxv[:, None, :] + byj[:, :, None] * pyv[:, None, :]
    a3 = (n2j[:, :, None] + p2p[:, None, :]) - 2.0 * gp
    e3 = gbj[:, :, None] - bp
    e23 = e3 + e3
    c3 = cj[:, :, None]

    xstar = -(e3 * crj[:, :, None]) * float(_T)
    sf = jnp.clip(jnp.floor(xstar), 0.0, float(_T - 1))
    s2 = jnp.minimum(sf + 1.0, float(_T - 1))
    w1 = sf / float(_T)
    w2 = s2 / float(_T)
    d1 = (c3 * w1 + e23) * w1 + a3
    d2 = (c3 * w2 + e23) * w2 + a3
    take1 = d1 <= d2
    dmin_j = jnp.where(take1, d1, d2)
    w_at = jnp.where(take1, w1, w2)

    runmin = jnp.min(dmin_j, axis=1)
    eq1 = dmin_j == runmin[:, None, :]
    jfirst = jnp.min(jnp.where(eq1, iota_j, _P), axis=1)
    onehot1 = iota_j == jfirst[:, None, :]
    tx3 = bxj[:, :, None] * w_at + gxrv[:, :, None]
    ty3 = byj[:, :, None] * w_at + gyrv[:, :, None]
    seltx = jnp.sum(jnp.where(onehot1, tx3, 0.0), axis=1)
    selty = jnp.sum(jnp.where(onehot1, ty3, 0.0), axis=1)

    valid1 = runmin <= _IGNORE_BOUND * _IGNORE_BOUND
    inv = 1.0 / _OFFSETS_STRIDE
    sl1 = _smooth_l1(oxv, (seltx - pxv) * inv) + _smooth_l1(oyv, (selty - pyv) * inv)
    acc[0] += jnp.sum(jnp.where(valid1, sl1, 0.0))
    acc[1] += jnp.sum(valid1.astype(jnp.float32))

    dx2 = pxv[:, :, None] - kxv[:, None, :]
    dy2 = pyv[:, :, None] - kyv[:, None, :]
    d2k = dx2 * dx2 + dy2 * dy2
    mn2 = jnp.min(d2k, axis=1)
    firstp = jnp.min(jnp.where(d2k == mn2[:, None, :], iota_j, _P), axis=1)
    onehot2 = iota_j == firstp[:, None, :]
    pselx = jnp.sum(jnp.where(onehot2, pxv[:, :, None], 0.0), axis=1)
    psely = jnp.sum(jnp.where(onehot2, pyv[:, :, None], 0.0), axis=1)
    oselx = jnp.sum(jnp.where(onehot2, oxv[:, :, None], 0.0), axis=1)
    osely = jnp.sum(jnp.where(onehot2, oyv[:, :, None], 0.0), axis=1)

    valid2 = mn2 <= _IGNORE_BOUND * _IGNORE_BOUND
    m2 = jnp.logical_and(mv > 0.0, valid2)
    sl2 = _smooth_l1(oselx, (kxv - pselx) * inv) + _smooth_l1(osely, (kyv - psely) * inv)
    acc[2] += jnp.sum(jnp.where(m2, sl2, 0.0))
    acc[3] += jnp.sum(m2.astype(jnp.float32))

    @pl.when(i == pl.num_programs(0) - 1)
    def _():
        denom1 = jnp.maximum(acc[1] * 2.0, 1.0)
        denom2 = jnp.maximum(acc[3] * 2.0, 1.0)
        out_ref[0, 0] = (acc[0] / denom1) * (1.0 - _KEY_ITEM_WEIGHT) + (
            acc[2] / denom2
        ) * _KEY_ITEM_WEIGHT


def kernel(pred_contours, pred_offsets, gt_contours, gt_key_points, gt_key_points_mask):
    px = pred_contours[..., 0]
    py = pred_contours[..., 1]
    ox = pred_offsets[..., 0]
    oy = pred_offsets[..., 1]
    gx = gt_contours[..., 0]
    gy = gt_contours[..., 1]
    gxr = jnp.roll(gx, 1, axis=1)
    gyr = jnp.roll(gy, 1, axis=1)
    kx = gt_key_points[..., 0]
    ky = gt_key_points[..., 1]
    m = gt_key_points_mask.astype(jnp.float32)

    out = pl.pallas_call(
        _dm_kernel,
        grid=(_N // _B,),
        in_specs=[pl.BlockSpec((_B, _P), lambda i: (i, 0))] * 11,
        out_specs=pl.BlockSpec(memory_space=pltpu.SMEM),
        out_shape=jax.ShapeDtypeStruct((1, 1), jnp.float32),
        scratch_shapes=[pltpu.SMEM((4,), jnp.float32)],
    )(px, py, ox, oy, gx, gy, gxr, gyr, kx, ky, m)
    return out[0, 0]


# speedup vs baseline: 5.0053x; 1.0474x over previous
import jax
import jax.numpy as jnp
from jax import lax
from jax.experimental import pallas as pl
from jax.experimental.pallas import tpu as pltpu

_N = 128
_P = 128
_T = 10
_OFFSETS_STRIDE = 4.0
_KEY_ITEM_WEIGHT = 0.5
_IGNORE_BOUND = 1000.0
_BETA = 1.0 / _OFFSETS_STRIDE
_B = 16


def _smooth_l1(pred, target):
    diff = jnp.abs(pred - target)
    return jnp.where(diff < _BETA, 0.5 * diff * diff / _BETA, diff - 0.5 * _BETA)


def _dm_kernel(px, py, ox, oy, gx, gy, gxr, gyr, kx, ky, m, out_ref, acc):
    i = pl.program_id(0)

    @pl.when(i == 0)
    def _():
        acc[0] = 0.0
        acc[1] = 0.0
        acc[2] = 0.0
        acc[3] = 0.0

    pxv = px[...]
    pyv = py[...]
    oxv = ox[...]
    oyv = oy[...]
    gxv = gx[...]
    gyv = gy[...]
    gxrv = gxr[...]
    gyrv = gyr[...]
    kxv = kx[...]
    kyv = ky[...]
    mv = m[...]

    iota_j = lax.broadcasted_iota(jnp.int32, (_B, _P, _P), 1)

    bxj = gxv - gxrv
    byj = gyv - gyrv
    cj = bxj * bxj + byj * byj
    crj = jnp.where(cj > 1e-30, 1.0 / cj, 0.0)
    gbj = gxrv * bxj + gyrv * byj
    n2j = gxrv * gxrv + gyrv * gyrv
    p2p = pxv * pxv + pyv * pyv

    gp = gxrv[:, :, None] * pxv[:, None, :] + gyrv[:, :, None] * pyv[:, None, :]
    bp = bxj[:, :, None] * pxv[:, None, :] + byj[:, :, None] * pyv[:, None, :]
    a3 = (n2j[:, :, None] + p2p[:, None, :]) - 2.0 * gp
    e3 = gbj[:, :, None] - bp
    e23 = e3 + e3
    c3 = cj[:, :, None]

    xstar = -(e3 * crj[:, :, None]) * float(_T)
    sf = jnp.clip(jnp.floor(xstar), 0.0, float(_T - 1))
    s2 = jnp.minimum(sf + 1.0, float(_T - 1))
    w1 = sf / float(_T)
    w2 = s2 / float(_T)
    d1 = (c3 * w1 + e23) * w1 + a3
    d2 = (c3 * w2 + e23) * w2 + a3
    take1 = d1 <= d2
    dmin_j = jnp.where(take1, d1, d2)
    w_at = jnp.where(take1, w1, w2)

    runmin = jnp.min(dmin_j, axis=1)
    eq1 = dmin_j == runmin[:, None, :]
    jfirst = jnp.min(jnp.where(eq1, iota_j, _P), axis=1)
    onehot1 = iota_j == jfirst[:, None, :]
    tx3 = bxj[:, :, None] * w_at + gxrv[:, :, None]
    ty3 = byj[:, :, None] * w_at + gyrv[:, :, None]
    seltx = jnp.sum(jnp.where(onehot1, tx3, 0.0), axis=1)
    selty = jnp.sum(jnp.where(onehot1, ty3, 0.0), axis=1)

    valid1 = runmin <= _IGNORE_BOUND * _IGNORE_BOUND
    inv = 1.0 / _OFFSETS_STRIDE
    sl1 = _smooth_l1(oxv, (seltx - pxv) * inv) + _smooth_l1(oyv, (selty - pyv) * inv)
    acc[0] += jnp.sum(jnp.where(valid1, sl1, 0.0))
    acc[1] += jnp.sum(valid1.astype(jnp.float32))

    dx2 = pxv[:, :, None] - kxv[:, None, :]
    dy2 = pyv[:, :, None] - kyv[:, None, :]
    d2k = dx2 * dx2 + dy2 * dy2
    mn2 = jnp.min(d2k, axis=1)
    firstp = jnp.min(jnp.where(d2k == mn2[:, None, :], iota_j, _P), axis=1)
    onehot2 = iota_j == firstp[:, None, :]
    pselx = jnp.sum(jnp.where(onehot2, pxv[:, :, None], 0.0), axis=1)
    psely = jnp.sum(jnp.where(onehot2, pyv[:, :, None], 0.0), axis=1)
    oselx = jnp.sum(jnp.where(onehot2, oxv[:, :, None], 0.0), axis=1)
    osely = jnp.sum(jnp.where(onehot2, oyv[:, :, None], 0.0), axis=1)

    valid2 = mn2 <= _IGNORE_BOUND * _IGNORE_BOUND
    m2 = jnp.logical_and(mv > 0.0, valid2)
    sl2 = _smooth_l1(oselx, (kxv - pselx) * inv) + _smooth_l1(osely, (kyv - psely) * inv)
    acc[2] += jnp.sum(jnp.where(m2, sl2, 0.0))
    acc[3] += jnp.sum(m2.astype(jnp.float32))

    @pl.when(i == pl.num_programs(0) - 1)
    def _():
        denom1 = jnp.maximum(acc[1] * 2.0, 1.0)
        denom2 = jnp.maximum(acc[3] * 2.0, 1.0)
        out_ref[0, 0] = (acc[0] / denom1) * (1.0 - _KEY_ITEM_WEIGHT) + (
            acc[2] / denom2
        ) * _KEY_ITEM_WEIGHT


def kernel(pred_contours, pred_offsets, gt_contours, gt_key_points, gt_key_points_mask):
    px = pred_contours[..., 0]
    py = pred_contours[..., 1]
    ox = pred_offsets[..., 0]
    oy = pred_offsets[..., 1]
    gx = gt_contours[..., 0]
    gy = gt_contours[..., 1]
    gxr = jnp.roll(gx, 1, axis=1)
    gyr = jnp.roll(gy, 1, axis=1)
    kx = gt_key_points[..., 0]
    ky = gt_key_points[..., 1]
    m = gt_key_points_mask.astype(jnp.float32)

    out = pl.pallas_call(
        _dm_kernel,
        grid=(_N // _B,),
        in_specs=[pl.BlockSpec((_B, _P), lambda i: (i, 0))] * 11,
        out_specs=pl.BlockSpec(memory_space=pltpu.SMEM),
        out_shape=jax.ShapeDtypeStruct((1, 1), jnp.float32),
        scratch_shapes=[pltpu.SMEM((4,), jnp.float32)],
    )(px, py, ox, oy, gx, gy, gxr, gyr, kx, ky, m)
    return out[0, 0]


# speedup vs baseline: 5.1396x; 1.0268x over previous
import jax
import jax.numpy as jnp
from jax import lax
from jax.experimental import pallas as pl
from jax.experimental.pallas import tpu as pltpu

_N = 128
_P = 128
_T = 10
_OFFSETS_STRIDE = 4.0
_KEY_ITEM_WEIGHT = 0.5
_IGNORE_BOUND = 1000.0
_BETA = 1.0 / _OFFSETS_STRIDE
_B = 32


def _smooth_l1(pred, target):
    diff = jnp.abs(pred - target)
    return jnp.where(diff < _BETA, 0.5 * diff * diff / _BETA, diff - 0.5 * _BETA)


def _dm_kernel(px, py, ox, oy, gx, gy, gxr, gyr, kx, ky, m, out_ref, acc):
    i = pl.program_id(0)

    @pl.when(i == 0)
    def _():
        acc[0] = 0.0
        acc[1] = 0.0
        acc[2] = 0.0
        acc[3] = 0.0

    pxv = px[...]
    pyv = py[...]
    oxv = ox[...]
    oyv = oy[...]
    gxv = gx[...]
    gyv = gy[...]
    gxrv = gxr[...]
    gyrv = gyr[...]
    kxv = kx[...]
    kyv = ky[...]
    mv = m[...]

    iota_j = lax.broadcasted_iota(jnp.int32, (_B, _P, _P), 1)

    bxj = gxv - gxrv
    byj = gyv - gyrv
    cj = bxj * bxj + byj * byj
    crj = jnp.where(cj > 1e-30, 1.0 / cj, 0.0)
    gbj = gxrv * bxj + gyrv * byj
    n2j = gxrv * gxrv + gyrv * gyrv
    p2p = pxv * pxv + pyv * pyv

    gp = gxrv[:, :, None] * pxv[:, None, :] + gyrv[:, :, None] * pyv[:, None, :]
    bp = bxj[:, :, None] * pxv[:, None, :] + byj[:, :, None] * pyv[:, None, :]
    a3 = (n2j[:, :, None] + p2p[:, None, :]) - 2.0 * gp
    e3 = gbj[:, :, None] - bp
    e23 = e3 + e3
    c3 = cj[:, :, None]

    xstar = -(e3 * crj[:, :, None]) * float(_T)
    sf = jnp.clip(jnp.floor(xstar), 0.0, float(_T - 1))
    s2 = jnp.minimum(sf + 1.0, float(_T - 1))
    w1 = sf / float(_T)
    w2 = s2 / float(_T)
    d1 = (c3 * w1 + e23) * w1 + a3
    d2 = (c3 * w2 + e23) * w2 + a3
    take1 = d1 <= d2
    dmin_j = jnp.where(take1, d1, d2)
    w_at = jnp.where(take1, w1, w2)

    runmin = jnp.min(dmin_j, axis=1)
    eq1 = dmin_j == runmin[:, None, :]
    jfirst = jnp.min(jnp.where(eq1, iota_j, _P), axis=1)
    onehot1 = iota_j == jfirst[:, None, :]
    tx3 = bxj[:, :, None] * w_at + gxrv[:, :, None]
    ty3 = byj[:, :, None] * w_at + gyrv[:, :, None]
    seltx = jnp.sum(jnp.where(onehot1, tx3, 0.0), axis=1)
    selty = jnp.sum(jnp.where(onehot1, ty3, 0.0), axis=1)

    valid1 = runmin <= _IGNORE_BOUND * _IGNORE_BOUND
    inv = 1.0 / _OFFSETS_STRIDE
    sl1 = _smooth_l1(oxv, (seltx - pxv) * inv) + _smooth_l1(oyv, (selty - pyv) * inv)
    acc[0] += jnp.sum(jnp.where(valid1, sl1, 0.0))
    acc[1] += jnp.sum(valid1.astype(jnp.float32))

    dx2 = pxv[:, :, None] - kxv[:, None, :]
    dy2 = pyv[:, :, None] - kyv[:, None, :]
    d2k = dx2 * dx2 + dy2 * dy2
    mn2 = jnp.min(d2k, axis=1)
    firstp = jnp.min(jnp.where(d2k == mn2[:, None, :], iota_j, _P), axis=1)
    onehot2 = iota_j == firstp[:, None, :]
    pselx = jnp.sum(jnp.where(onehot2, pxv[:, :, None], 0.0), axis=1)
    psely = jnp.sum(jnp.where(onehot2, pyv[:, :, None], 0.0), axis=1)
    oselx = jnp.sum(jnp.where(onehot2, oxv[:, :, None], 0.0), axis=1)
    osely = jnp.sum(jnp.where(onehot2, oyv[:, :, None], 0.0), axis=1)

    valid2 = mn2 <= _IGNORE_BOUND * _IGNORE_BOUND
    m2 = jnp.logical_and(mv > 0.0, valid2)
    sl2 = _smooth_l1(oselx, (kxv - pselx) * inv) + _smooth_l1(osely, (kyv - psely) * inv)
    acc[2] += jnp.sum(jnp.where(m2, sl2, 0.0))
    acc[3] += jnp.sum(m2.astype(jnp.float32))

    @pl.when(i == pl.num_programs(0) - 1)
    def _():
        denom1 = jnp.maximum(acc[1] * 2.0, 1.0)
        denom2 = jnp.maximum(acc[3] * 2.0, 1.0)
        out_ref[0, 0] = (acc[0] / denom1) * (1.0 - _KEY_ITEM_WEIGHT) + (
            acc[2] / denom2
        ) * _KEY_ITEM_WEIGHT


def kernel(pred_contours, pred_offsets, gt_contours, gt_key_points, gt_key_points_mask):
    px = pred_contours[..., 0]
    py = pred_contours[..., 1]
    ox = pred_offsets[..., 0]
    oy = pred_offsets[..., 1]
    gx = gt_contours[..., 0]
    gy = gt_contours[..., 1]
    gxr = jnp.roll(gx, 1, axis=1)
    gyr = jnp.roll(gy, 1, axis=1)
    kx = gt_key_points[..., 0]
    ky = gt_key_points[..., 1]
    m = gt_key_points_mask.astype(jnp.float32)

    out = pl.pallas_call(
        _dm_kernel,
        grid=(_N // _B,),
        in_specs=[pl.BlockSpec((_B, _P), lambda i: (i, 0))] * 11,
        out_specs=pl.BlockSpec(memory_space=pltpu.SMEM),
        out_shape=jax.ShapeDtypeStruct((1, 1), jnp.float32),
        scratch_shapes=[pltpu.SMEM((4,), jnp.float32)],
    )(px, py, ox, oy, gx, gy, gxr, gyr, kx, ky, m)
    return out[0, 0]


# speedup vs baseline: 9.0670x; 1.7641x over previous
import jax
import jax.numpy as jnp
from jax.experimental import pallas as pl
from jax.experimental.pallas import tpu as pltpu

_N = 128
_P = 128
_T = 10
_OFFSETS_STRIDE = 4.0
_KEY_ITEM_WEIGHT = 0.5
_IGNORE_BOUND = 1000.0
_BETA = 1.0 / _OFFSETS_STRIDE
_JC = 8


def _smooth_l1(pred, target):
    diff = jnp.abs(pred - target)
    return jnp.where(diff < _BETA, 0.5 * diff * diff / _BETA, diff - 0.5 * _BETA)


def _dm_kernel(
    pxf, pyf, oxf, oyf, kxf, kyf, mf,
    gxc, gyc, gxrc, gyrc, pxc, pyc, oxc, oyc,
    out_ref,
    runmin, seltx, selty, mn2, pselx, psely, oselx, osely,
):
    i = pl.program_id(0)

    @pl.when(i == 0)
    def _():
        runmin[...] = jnp.full((_P, _N), jnp.inf, jnp.float32)
        mn2[...] = jnp.full((_P, _N), jnp.inf, jnp.float32)

    gxr3 = gxrc[...][:, None, :]
    gyr3 = gyrc[...][:, None, :]
    bx3 = gxc[...][:, None, :] - gxr3
    by3 = gyc[...][:, None, :] - gyr3
    c3 = bx3 * bx3 + by3 * by3
    ncr = jnp.where(c3 > 1e-30, -float(_T) / c3, 0.0)

    px3 = pxf[...][None, :, :]
    py3 = pyf[...][None, :, :]
    dx = gxr3 - px3
    dy = gyr3 - py3
    a3 = dx * dx + dy * dy
    e3 = dx * bx3 + dy * by3
    e23 = e3 + e3

    xs = e3 * ncr
    sf = jnp.clip(jnp.floor(xs), 0.0, float(_T - 1))
    s2 = jnp.minimum(sf + 1.0, float(_T - 1))
    w1 = sf * (1.0 / _T)
    w2 = s2 * (1.0 / _T)
    d1 = (c3 * w1 + e23) * w1 + a3
    d2 = (c3 * w2 + e23) * w2 + a3
    take1 = d1 <= d2
    dmin = jnp.where(take1, d1, d2)
    w_at = jnp.where(take1, w1, w2)
    tx3 = bx3 * w_at + gxr3
    ty3 = by3 * w_at + gyr3

    rm = runmin[...]
    sx = seltx[...]
    sy = selty[...]
    for jj in range(_JC):
        upd = dmin[jj] < rm
        rm = jnp.where(upd, dmin[jj], rm)
        sx = jnp.where(upd, tx3[jj], sx)
        sy = jnp.where(upd, ty3[jj], sy)
    runmin[...] = rm
    seltx[...] = sx
    selty[...] = sy

    kx3 = kxf[...][None, :, :]
    ky3 = kyf[...][None, :, :]
    pxr = pxc[...]
    pyr = pyc[...]
    oxr = oxc[...]
    oyr = oyc[...]
    dx2 = pxr[:, None, :] - kx3
    dy2 = pyr[:, None, :] - ky3
    dd2 = dx2 * dx2 + dy2 * dy2

    m2v = mn2[...]
    qx = pselx[...]
    qy = psely[...]
    rx = oselx[...]
    ry = osely[...]
    for jj in range(_JC):
        upd = dd2[jj] < m2v
        m2v = jnp.where(upd, dd2[jj], m2v)
        qx = jnp.where(upd, pxr[jj][None, :], qx)
        qy = jnp.where(upd, pyr[jj][None, :], qy)
        rx = jnp.where(upd, oxr[jj][None, :], rx)
        ry = jnp.where(upd, oyr[jj][None, :], ry)
    mn2[...] = m2v
    pselx[...] = qx
    psely[...] = qy
    oselx[...] = rx
    osely[...] = ry

    @pl.when(i == pl.num_programs(0) - 1)
    def _():
        inv = 1.0 / _OFFSETS_STRIDE
        bound = _IGNORE_BOUND * _IGNORE_BOUND
        valid1 = rm <= bound
        sl1 = _smooth_l1(oxf[...], (sx - pxf[...]) * inv) + _smooth_l1(
            oyf[...], (sy - pyf[...]) * inv
        )
        s1 = jnp.sum(jnp.where(valid1, sl1, 0.0))
        c1 = jnp.sum(valid1.astype(jnp.float32))

        valid2 = m2v <= bound
        mk = jnp.logical_and(mf[...] > 0.0, valid2)
        sl2 = _smooth_l1(rx, (kxf[...] - qx) * inv) + _smooth_l1(
            ry, (kyf[...] - qy) * inv
        )
        s2s = jnp.sum(jnp.where(mk, sl2, 0.0))
        c2 = jnp.sum(mk.astype(jnp.float32))

        denom1 = jnp.maximum(c1 * 2.0, 1.0)
        denom2 = jnp.maximum(c2 * 2.0, 1.0)
        out_ref[0, 0] = (s1 / denom1) * (1.0 - _KEY_ITEM_WEIGHT) + (
            s2s / denom2
        ) * _KEY_ITEM_WEIGHT


def kernel(pred_contours, pred_offsets, gt_contours, gt_key_points, gt_key_points_mask):
    px = pred_contours[..., 0].T
    py = pred_contours[..., 1].T
    ox = pred_offsets[..., 0].T
    oy = pred_offsets[..., 1].T
    gx = gt_contours[..., 0].T
    gy = gt_contours[..., 1].T
    gxr = jnp.roll(gx, 1, axis=0)
    gyr = jnp.roll(gy, 1, axis=0)
    kx = gt_key_points[..., 0].T
    ky = gt_key_points[..., 1].T
    m = gt_key_points_mask.astype(jnp.float32).T

    full = pl.BlockSpec((_P, _N), lambda i: (0, 0))
    chunk = pl.BlockSpec((_JC, _N), lambda i: (i, 0))
    out = pl.pallas_call(
        _dm_kernel,
        grid=(_P // _JC,),
        in_specs=[full] * 7 + [chunk] * 8,
        out_specs=pl.BlockSpec(memory_space=pltpu.SMEM),
        out_shape=jax.ShapeDtypeStruct((1, 1), jnp.float32),
        scratch_shapes=[pltpu.VMEM((_P, _N), jnp.float32)] * 8,
    )(px, py, ox, oy, kx, ky, m, gx, gy, gxr, gyr, px, py, ox, oy)
    return out[0, 0]
